# baseline (speedup 1.0000x reference)
import numpy as np
import jax
import jax.numpy as jnp
from jax import lax
from jax.experimental import pallas as pl
from jax.experimental.pallas import tpu as pltpu

_IMG = 416
_NC = 80
_NOUT = 5 + _NC
_N_SAMPLE = 256.0
_N_POS = 128.0


def _static_geometry():
    anchor_sets = {
        32: [(116, 90), (156, 198), (373, 326)],
        16: [(30, 61), (62, 45), (59, 119)],
        8: [(10, 13), (16, 30), (33, 23)],
    }
    segs = []
    cols = []
    off = 0
    for stride in (8, 16, 32):
        g = _IMG // stride
        centers = ((np.arange(g) + 0.5) * stride).astype(np.float32)
        for (aw, ah) in anchor_sets[stride]:
            okx = (centers - aw / 2.0 >= 0) & (centers + aw / 2.0 <= _IMG)
            oky = (centers - ah / 2.0 >= 0) & (centers + ah / 2.0 <= _IMG)
            xi = np.nonzero(okx)[0]
            yi = np.nonzero(oky)[0]
            x0, x1 = int(xi[0]), int(xi[-1])
            y0, y1 = int(yi[0]), int(yi[-1])
            segs.append((off, g, y0, y1, x0, x1))
            cxv = np.tile(centers[x0:x1 + 1], y1 - y0 + 1)
            cyv = np.repeat(centers[y0:y1 + 1], x1 - x0 + 1)
            n = cxv.shape[0]
            cols.append(np.stack([
                cxv, cyv,
                np.full((n,), aw, np.float32),
                np.full((n,), ah, np.float32),
                np.full((n,), stride, np.float32),
            ], 1))
            off += g * g
    A = np.concatenate(cols, 0).astype(np.float32)
    cx, cy, w, h, s = A[:, 0], A[:, 1], A[:, 2], A[:, 3], A[:, 4]
    half = np.float32(2.0)
    const = np.stack([
        cx, cy, w, h,
        cx - w / half, cy - h / half, cx + w / half, cy + h / half,
        s,
    ], 1).astype(np.float32)
    return segs, np.ascontiguousarray(const.T)


_SEGS, _CONST_T = _static_geometry()
_NV = _CONST_T.shape[1]


def _body(p_hbm, gt5_ref, const_ref, preds_hbm, loss_ref,
          p_vmem, pv_vmem, in_sem, out_sem):
    f32 = jnp.float32
    N = _NV
    cp_in = pltpu.make_async_copy(p_hbm.at[0], p_vmem, in_sem)
    cp_in.start()
    gt5 = gt5_ref[...]
    C = const_ref[...]

    acx, acy, aw, ah = C[0:1], C[1:2], C[2:3], C[3:4]
    ax1, ay1, ax2, ay2 = C[4:5], C[5:6], C[6:7], C[7:8]
    astr = C[8:9]

    gcx, gcy = gt5[:, 0:1], gt5[:, 1:2]
    gw, gh = gt5[:, 2:3], gt5[:, 3:4]
    gx1 = gcx - gw / 2.0
    gy1 = gcy - gh / 2.0
    gx2 = gcx + gw / 2.0
    gy2 = gcy + gh / 2.0

    iw = jnp.maximum(jnp.minimum(ax2, gx2) - jnp.maximum(ax1, gx1), 0.0)
    ih = jnp.maximum(jnp.minimum(ay2, gy2) - jnp.maximum(ay1, gy1), 0.0)
    inter = iw * ih
    area_a = (ax2 - ax1) * (ay2 - ay1)
    area_b = (gx2 - gx1) * (gy2 - gy1)
    iou = inter / (area_a + area_b - inter)

    siota = lax.broadcasted_iota(jnp.int32, (32, N), 0)
    liota = lax.broadcasted_iota(jnp.int32, (32, N), 1)

    amax = jnp.max(iou, axis=0, keepdims=True)
    amax_idx = jnp.min(jnp.where(iou == amax, siota, 32), axis=0,
                       keepdims=True)
    gmax = jnp.max(iou, axis=1, keepdims=True)
    gt_idx = jnp.min(jnp.where(iou == gmax, liota, N), axis=1,
                     keepdims=True)
    matched = jnp.any(liota == gt_idx, axis=0, keepdims=True)

    labels = jnp.where(amax >= 0.5, 1.0, -1.0)
    labels = jnp.where(amax < 0.4, 0.0, labels)
    labels = jnp.where(matched, 1.0, labels)

    is_pos = labels == 1.0
    is_neg = labels == 0.0

    flags = jnp.concatenate(
        [is_pos.astype(f32), is_neg.astype(f32)], 0)
    c = flags
    k = 1
    while k < N:
        c = c + jnp.concatenate(
            [jnp.zeros((2, k), f32), c[:, :N - k]], 1)
        k *= 2
    pos_rank = c[0:1, :] - 1.0
    neg_rank = c[1:2, :] - 1.0

    n_pos_actual = jnp.sum(flags[0:1, :])
    labels = jnp.where(is_pos & (pos_rank < n_pos_actual - _N_POS),
                       -1.0, labels)
    n_neg_actual = jnp.sum(flags[1:2, :])
    n_neg_req = _N_SAMPLE - jnp.sum((labels == 1.0).astype(f32))
    labels = jnp.where(is_neg & (neg_rank < n_neg_actual - n_neg_req),
                       -1.0, labels)

    amatch = (amax_idx == siota).astype(f32)
    g5 = lax.dot_general(gt5, amatch, (((0,), (0,)), ((), ())),
                         precision=lax.Precision.HIGHEST)
    agb_cx, agb_cy = g5[0:1], g5[1:2]
    agb_w, agb_h = g5[2:3], g5[3:4]
    cls = g5[4:5]

    tx = (agb_cx - acx) / astr
    ty = (agb_cy - acy) / astr
    tw = jnp.log(agb_w / aw)
    th = jnp.log(agb_h / ah)

    obj = labels == 1.0
    nonobj = labels == 0.0
    n_obj = jnp.sum(obj.astype(f32))
    n_nonobj = jnp.sum(nonobj.astype(f32))

    cp_in.wait()
    pieces = []
    for (off, g, y0, y1, x0, x1) in _SEGS:
        w = x1 - x0 + 1
        for y in range(y0, y1 + 1):
            s = off + y * g + x0
            pieces.append(p_vmem[s:s + w, :])
    pv = jnp.concatenate(pieces, 0)
    pv_vmem[...] = pv
    cp_out = pltpu.make_async_copy(pv_vmem, preds_hbm, out_sem)
    cp_out.start()
    pvT = pv.T

    sq_sum = ((pvT[0:1] - tx) ** 2 + (pvT[1:2] - ty) ** 2
              + (pvT[2:3] - tw) ** 2 + (pvT[3:4] - th) ** 2)
    mse_sum = jnp.sum(jnp.where(obj, sq_sum, 0.0))

    p4 = pvT[4:5]
    logp4 = jnp.maximum(jnp.log(p4), -100.0)
    log1mp4 = jnp.maximum(jnp.log(1.0 - p4), -100.0)
    conf_obj_sum = jnp.sum(jnp.where(obj, -logp4, 0.0))
    conf_nonobj_sum = jnp.sum(jnp.where(nonobj, -log1mp4, 0.0))

    pcl = pvT[5:_NOUT]
    log1mpc = jnp.maximum(jnp.log(1.0 - pcl), -100.0)
    onehot = (lax.broadcasted_iota(jnp.int32, (_NC, N), 0)
              == cls.astype(jnp.int32))
    p_at = jnp.sum(jnp.where(onehot, pcl, 0.0), axis=0, keepdims=True)
    l1_at = jnp.sum(jnp.where(onehot, log1mpc, 0.0), axis=0, keepdims=True)
    lp_at = jnp.maximum(jnp.log(p_at), -100.0)
    row_l1_sum = jnp.sum(log1mpc, axis=0, keepdims=True)
    cls_row = -row_l1_sum + l1_at - lp_at
    cls_sum = jnp.sum(jnp.where(obj, cls_row, 0.0))

    total = (mse_sum / n_obj
             + conf_obj_sum / n_obj
             + conf_nonobj_sum / n_nonobj
             + cls_sum / (n_obj * float(_NC)))
    loss_ref[...] = jnp.broadcast_to(total, (1, 1))
    cp_out.wait()


def kernel(predictions, gt_boxes, gt_labels):
    gt5 = jnp.concatenate(
        [gt_boxes, gt_labels.astype(jnp.float32)[:, None]], 1)
    const = jnp.asarray(_CONST_T)

    preds, loss = pl.pallas_call(
        _body,
        out_shape=(
            jax.ShapeDtypeStruct((_NV, _NOUT), jnp.float32),
            jax.ShapeDtypeStruct((1, 1), jnp.float32),
        ),
        in_specs=[
            pl.BlockSpec(memory_space=pl.ANY),
            pl.BlockSpec((32, 5), lambda: (0, 0)),
            pl.BlockSpec((9, _NV), lambda: (0, 0)),
        ],
        out_specs=(
            pl.BlockSpec(memory_space=pl.ANY),
            pl.BlockSpec((1, 1), lambda: (0, 0)),
        ),
        scratch_shapes=[
            pltpu.VMEM((10647, _NOUT), jnp.float32),
            pltpu.VMEM((_NV, _NOUT), jnp.float32),
            pltpu.SemaphoreType.DMA,
            pltpu.SemaphoreType.DMA,
        ],
    )(predictions, gt5, const)
    return preds, loss.reshape(())


# speedup vs baseline: 1.6228x; 1.6228x over previous
import numpy as np
import jax
import jax.numpy as jnp
from jax import lax
from jax.experimental import pallas as pl
from jax.experimental.pallas import tpu as pltpu

_IMG = 416
_NC = 80
_NOUT = 5 + _NC
_N_SAMPLE = 256.0
_N_POS = 128.0


def _static_geometry():
    anchor_sets = {
        32: [(116, 90), (156, 198), (373, 326)],
        16: [(30, 61), (62, 45), (59, 119)],
        8: [(10, 13), (16, 30), (33, 23)],
    }
    segs = []
    cols = []
    off = 0
    for stride in (8, 16, 32):
        g = _IMG // stride
        centers = ((np.arange(g) + 0.5) * stride).astype(np.float32)
        for (aw, ah) in anchor_sets[stride]:
            okx = (centers - aw / 2.0 >= 0) & (centers + aw / 2.0 <= _IMG)
            oky = (centers - ah / 2.0 >= 0) & (centers + ah / 2.0 <= _IMG)
            xi = np.nonzero(okx)[0]
            yi = np.nonzero(oky)[0]
            x0, x1 = int(xi[0]), int(xi[-1])
            y0, y1 = int(yi[0]), int(yi[-1])
            segs.append((off, g, y0, y1, x0, x1))
            cxv = np.tile(centers[x0:x1 + 1], y1 - y0 + 1)
            cyv = np.repeat(centers[y0:y1 + 1], x1 - x0 + 1)
            n = cxv.shape[0]
            cols.append(np.stack([
                cxv, cyv,
                np.full((n,), aw, np.float32),
                np.full((n,), ah, np.float32),
                np.full((n,), stride, np.float32),
            ], 1))
            off += g * g
    A = np.concatenate(cols, 0).astype(np.float32)
    cx, cy, w, h, s = A[:, 0], A[:, 1], A[:, 2], A[:, 3], A[:, 4]
    half = np.float32(2.0)
    const = np.stack([
        cx, cy, w, h,
        cx - w / half, cy - h / half, cx + w / half, cy + h / half,
        s,
    ], 1).astype(np.float32)
    return segs, np.ascontiguousarray(const.T)


_SEGS, _CONST_T = _static_geometry()
_NV = _CONST_T.shape[1]


def _body(p_hbm, gt5_ref, const_ref, preds_hbm, loss_ref,
          p_vmem, pv_vmem, in_sem, out_sem):
    f32 = jnp.float32
    N = _NV
    cp_in = pltpu.make_async_copy(p_hbm, p_vmem, in_sem)
    cp_in.start()
    gt5 = gt5_ref[...]
    C = const_ref[...]

    acx, acy, aw, ah = C[0:1], C[1:2], C[2:3], C[3:4]
    ax1, ay1, ax2, ay2 = C[4:5], C[5:6], C[6:7], C[7:8]
    astr = C[8:9]

    gcx, gcy = gt5[:, 0:1], gt5[:, 1:2]
    gw, gh = gt5[:, 2:3], gt5[:, 3:4]
    gx1 = gcx - gw / 2.0
    gy1 = gcy - gh / 2.0
    gx2 = gcx + gw / 2.0
    gy2 = gcy + gh / 2.0

    iw = jnp.maximum(jnp.minimum(ax2, gx2) - jnp.maximum(ax1, gx1), 0.0)
    ih = jnp.maximum(jnp.minimum(ay2, gy2) - jnp.maximum(ay1, gy1), 0.0)
    inter = iw * ih
    area_a = (ax2 - ax1) * (ay2 - ay1)
    area_b = (gx2 - gx1) * (gy2 - gy1)
    iou = inter / (area_a + area_b - inter)

    siota = lax.broadcasted_iota(jnp.int32, (32, N), 0)
    liota = lax.broadcasted_iota(jnp.int32, (32, N), 1)

    amax = jnp.max(iou, axis=0, keepdims=True)
    amax_idx = jnp.min(jnp.where(iou == amax, siota, 32), axis=0,
                       keepdims=True)
    gmax = jnp.max(iou, axis=1, keepdims=True)
    gt_idx = jnp.min(jnp.where(iou == gmax, liota, N), axis=1,
                     keepdims=True)
    matched = jnp.any(liota == gt_idx, axis=0, keepdims=True)

    labels = jnp.where(amax >= 0.5, 1.0, -1.0)
    labels = jnp.where(amax < 0.4, 0.0, labels)
    labels = jnp.where(matched, 1.0, labels)

    is_pos = labels == 1.0
    is_neg = labels == 0.0

    flags = jnp.concatenate(
        [is_pos.astype(f32), is_neg.astype(f32)], 0)
    c = flags
    k = 1
    while k < N:
        c = c + jnp.concatenate(
            [jnp.zeros((2, k), f32), c[:, :N - k]], 1)
        k *= 2
    pos_rank = c[0:1, :] - 1.0
    neg_rank = c[1:2, :] - 1.0

    n_pos_actual = jnp.sum(flags[0:1, :])
    labels = jnp.where(is_pos & (pos_rank < n_pos_actual - _N_POS),
                       -1.0, labels)
    n_neg_actual = jnp.sum(flags[1:2, :])
    n_neg_req = _N_SAMPLE - jnp.sum((labels == 1.0).astype(f32))
    labels = jnp.where(is_neg & (neg_rank < n_neg_actual - n_neg_req),
                       -1.0, labels)

    amatch = (amax_idx == siota).astype(f32)
    g5 = lax.dot_general(gt5, amatch, (((0,), (0,)), ((), ())),
                         precision=lax.Precision.HIGHEST)
    agb_cx, agb_cy = g5[0:1], g5[1:2]
    agb_w, agb_h = g5[2:3], g5[3:4]
    cls = g5[4:5]

    tx = (agb_cx - acx) / astr
    ty = (agb_cy - acy) / astr
    tw = jnp.log(agb_w / aw)
    th = jnp.log(agb_h / ah)

    obj = labels == 1.0
    nonobj = labels == 0.0
    n_obj = jnp.sum(obj.astype(f32))
    n_nonobj = jnp.sum(nonobj.astype(f32))

    cp_in.wait()
    pieces = []
    for (off, g, y0, y1, x0, x1) in _SEGS:
        w = x1 - x0 + 1
        for y in range(y0, y1 + 1):
            s = off + y * g + x0
            pieces.append(p_vmem[:, s:s + w])
    pvT = jnp.concatenate(pieces, 1)
    pv_vmem[...] = pvT
    cp_out = pltpu.make_async_copy(pv_vmem, preds_hbm, out_sem)
    cp_out.start()

    sq_sum = ((pvT[0:1] - tx) ** 2 + (pvT[1:2] - ty) ** 2
              + (pvT[2:3] - tw) ** 2 + (pvT[3:4] - th) ** 2)
    mse_sum = jnp.sum(jnp.where(obj, sq_sum, 0.0))

    p4 = pvT[4:5]
    logp4 = jnp.maximum(jnp.log(p4), -100.0)
    log1mp4 = jnp.maximum(jnp.log(1.0 - p4), -100.0)
    conf_obj_sum = jnp.sum(jnp.where(obj, -logp4, 0.0))
    conf_nonobj_sum = jnp.sum(jnp.where(nonobj, -log1mp4, 0.0))

    pcl = pvT[5:_NOUT]
    log1mpc = jnp.maximum(jnp.log(1.0 - pcl), -100.0)
    onehot = (lax.broadcasted_iota(jnp.int32, (_NC, N), 0)
              == cls.astype(jnp.int32))
    p_at = jnp.sum(jnp.where(onehot, pcl, 0.0), axis=0, keepdims=True)
    l1_at = jnp.sum(jnp.where(onehot, log1mpc, 0.0), axis=0, keepdims=True)
    lp_at = jnp.maximum(jnp.log(p_at), -100.0)
    row_l1_sum = jnp.sum(log1mpc, axis=0, keepdims=True)
    cls_row = -row_l1_sum + l1_at - lp_at
    cls_sum = jnp.sum(jnp.where(obj, cls_row, 0.0))

    total = (mse_sum / n_obj
             + conf_obj_sum / n_obj
             + conf_nonobj_sum / n_nonobj
             + cls_sum / (n_obj * float(_NC)))
    loss_ref[...] = jnp.broadcast_to(total, (1, 1))
    cp_out.wait()


def kernel(predictions, gt_boxes, gt_labels):
    pT = predictions[0].T
    gt5 = jnp.concatenate(
        [gt_boxes, gt_labels.astype(jnp.float32)[:, None]], 1)
    const = jnp.asarray(_CONST_T)

    predsT, loss = pl.pallas_call(
        _body,
        out_shape=(
            jax.ShapeDtypeStruct((_NOUT, _NV), jnp.float32),
            jax.ShapeDtypeStruct((1, 1), jnp.float32),
        ),
        in_specs=[
            pl.BlockSpec(memory_space=pl.ANY),
            pl.BlockSpec((32, 5), lambda: (0, 0)),
            pl.BlockSpec((9, _NV), lambda: (0, 0)),
        ],
        out_specs=(
            pl.BlockSpec(memory_space=pl.ANY),
            pl.BlockSpec((1, 1), lambda: (0, 0)),
        ),
        scratch_shapes=[
            pltpu.VMEM((_NOUT, 10647), jnp.float32),
            pltpu.VMEM((_NOUT, _NV), jnp.float32),
            pltpu.SemaphoreType.DMA,
            pltpu.SemaphoreType.DMA,
        ],
    )(pT, gt5, const)
    return predsT.T, loss.reshape(())


# speedup vs baseline: 1.6894x; 1.0411x over previous
import numpy as np
import jax
import jax.numpy as jnp
from jax import lax
from jax.experimental import pallas as pl
from jax.experimental.pallas import tpu as pltpu

_IMG = 416
_NC = 80
_NOUT = 5 + _NC
_N_SAMPLE = 256.0
_N_POS = 128.0


def _static_geometry():
    anchor_sets = {
        32: [(116, 90), (156, 198), (373, 326)],
        16: [(30, 61), (62, 45), (59, 119)],
        8: [(10, 13), (16, 30), (33, 23)],
    }
    segs = []
    cols = []
    off = 0
    for stride in (8, 16, 32):
        g = _IMG // stride
        centers = ((np.arange(g) + 0.5) * stride).astype(np.float32)
        for (aw, ah) in anchor_sets[stride]:
            okx = (centers - aw / 2.0 >= 0) & (centers + aw / 2.0 <= _IMG)
            oky = (centers - ah / 2.0 >= 0) & (centers + ah / 2.0 <= _IMG)
            xi = np.nonzero(okx)[0]
            yi = np.nonzero(oky)[0]
            x0, x1 = int(xi[0]), int(xi[-1])
            y0, y1 = int(yi[0]), int(yi[-1])
            segs.append((off, g, y0, y1, x0, x1))
            cxv = np.tile(centers[x0:x1 + 1], y1 - y0 + 1)
            cyv = np.repeat(centers[y0:y1 + 1], x1 - x0 + 1)
            n = cxv.shape[0]
            cols.append(np.stack([
                cxv, cyv,
                np.full((n,), aw, np.float32),
                np.full((n,), ah, np.float32),
                np.full((n,), stride, np.float32),
            ], 1))
            off += g * g
    A = np.concatenate(cols, 0).astype(np.float32)
    cx, cy, w, h, s = A[:, 0], A[:, 1], A[:, 2], A[:, 3], A[:, 4]
    half = np.float32(2.0)
    const = np.stack([
        cx, cy, w, h,
        cx - w / half, cy - h / half, cx + w / half, cy + h / half,
        s,
    ], 1).astype(np.float32)
    return segs, np.ascontiguousarray(const.T)


_SEGS, _CONST_T = _static_geometry()
_NV = _CONST_T.shape[1]


def _body(p_hbm, gtbT_ref, glab_ref, const_ref, preds_hbm, loss_ref,
          p_vmem, pv_vmem, in_sem, out_sem):
    f32 = jnp.float32
    N = _NV
    cp_in = pltpu.make_async_copy(p_hbm, p_vmem, in_sem)
    cp_in.start()
    gt5 = jnp.concatenate(
        [gtbT_ref[...].T, glab_ref[...].astype(f32)], 1)
    C = const_ref[...]

    acx, acy, aw, ah = C[0:1], C[1:2], C[2:3], C[3:4]
    ax1, ay1, ax2, ay2 = C[4:5], C[5:6], C[6:7], C[7:8]
    astr = C[8:9]

    gcx, gcy = gt5[:, 0:1], gt5[:, 1:2]
    gw, gh = gt5[:, 2:3], gt5[:, 3:4]
    gx1 = gcx - gw / 2.0
    gy1 = gcy - gh / 2.0
    gx2 = gcx + gw / 2.0
    gy2 = gcy + gh / 2.0

    iw = jnp.maximum(jnp.minimum(ax2, gx2) - jnp.maximum(ax1, gx1), 0.0)
    ih = jnp.maximum(jnp.minimum(ay2, gy2) - jnp.maximum(ay1, gy1), 0.0)
    inter = iw * ih
    area_a = (ax2 - ax1) * (ay2 - ay1)
    area_b = (gx2 - gx1) * (gy2 - gy1)
    iou = inter / (area_a + area_b - inter)

    siota = lax.broadcasted_iota(jnp.int32, (32, N), 0)
    liota = lax.broadcasted_iota(jnp.int32, (32, N), 1)

    amax = jnp.max(iou, axis=0, keepdims=True)
    amax_idx = jnp.min(jnp.where(iou == amax, siota, 32), axis=0,
                       keepdims=True)
    gmax = jnp.max(iou, axis=1, keepdims=True)
    gt_idx = jnp.min(jnp.where(iou == gmax, liota, N), axis=1,
                     keepdims=True)
    matched = jnp.any(liota == gt_idx, axis=0, keepdims=True)

    labels = jnp.where(amax >= 0.5, 1.0, -1.0)
    labels = jnp.where(amax < 0.4, 0.0, labels)
    labels = jnp.where(matched, 1.0, labels)

    is_pos = labels == 1.0
    is_neg = labels == 0.0

    flags = jnp.concatenate(
        [is_pos.astype(f32), is_neg.astype(f32)], 0)
    c = flags
    k = 1
    while k < N:
        c = c + jnp.concatenate(
            [jnp.zeros((2, k), f32), c[:, :N - k]], 1)
        k *= 2
    pos_rank = c[0:1, :] - 1.0
    neg_rank = c[1:2, :] - 1.0

    n_pos_actual = jnp.sum(flags[0:1, :])
    labels = jnp.where(is_pos & (pos_rank < n_pos_actual - _N_POS),
                       -1.0, labels)
    n_neg_actual = jnp.sum(flags[1:2, :])
    n_neg_req = _N_SAMPLE - jnp.sum((labels == 1.0).astype(f32))
    labels = jnp.where(is_neg & (neg_rank < n_neg_actual - n_neg_req),
                       -1.0, labels)

    amatch = (amax_idx == siota).astype(f32)
    g5 = lax.dot_general(gt5, amatch, (((0,), (0,)), ((), ())),
                         precision=lax.Precision.HIGHEST)
    agb_cx, agb_cy = g5[0:1], g5[1:2]
    agb_w, agb_h = g5[2:3], g5[3:4]
    cls = g5[4:5]

    tx = (agb_cx - acx) / astr
    ty = (agb_cy - acy) / astr
    tw = jnp.log(agb_w / aw)
    th = jnp.log(agb_h / ah)

    obj = labels == 1.0
    nonobj = labels == 0.0
    n_obj = jnp.sum(obj.astype(f32))
    n_nonobj = jnp.sum(nonobj.astype(f32))

    cp_in.wait()
    pieces = []
    for (off, g, y0, y1, x0, x1) in _SEGS:
        w = x1 - x0 + 1
        for y in range(y0, y1 + 1):
            s = off + y * g + x0
            pieces.append(p_vmem[:, s:s + w])
    pvT = jnp.concatenate(pieces, 1)
    pv_vmem[...] = pvT
    cp_out = pltpu.make_async_copy(pv_vmem, preds_hbm, out_sem)
    cp_out.start()

    sq_sum = ((pvT[0:1] - tx) ** 2 + (pvT[1:2] - ty) ** 2
              + (pvT[2:3] - tw) ** 2 + (pvT[3:4] - th) ** 2)
    mse_sum = jnp.sum(jnp.where(obj, sq_sum, 0.0))

    p4 = pvT[4:5]
    logp4 = jnp.maximum(jnp.log(p4), -100.0)
    log1mp4 = jnp.maximum(jnp.log(1.0 - p4), -100.0)
    conf_obj_sum = jnp.sum(jnp.where(obj, -logp4, 0.0))
    conf_nonobj_sum = jnp.sum(jnp.where(nonobj, -log1mp4, 0.0))

    pcl = pvT[5:_NOUT]
    log1mpc = jnp.maximum(jnp.log(1.0 - pcl), -100.0)
    onehot = (lax.broadcasted_iota(jnp.int32, (_NC, N), 0)
              == cls.astype(jnp.int32))
    p_at = jnp.sum(jnp.where(onehot, pcl, 0.0), axis=0, keepdims=True)
    l1_at = jnp.sum(jnp.where(onehot, log1mpc, 0.0), axis=0, keepdims=True)
    lp_at = jnp.maximum(jnp.log(p_at), -100.0)
    row_l1_sum = jnp.sum(log1mpc, axis=0, keepdims=True)
    cls_row = -row_l1_sum + l1_at - lp_at
    cls_sum = jnp.sum(jnp.where(obj, cls_row, 0.0))

    total = (mse_sum / n_obj
             + conf_obj_sum / n_obj
             + conf_nonobj_sum / n_nonobj
             + cls_sum / (n_obj * float(_NC)))
    loss_ref[...] = jnp.broadcast_to(total, (1, 1))
    cp_out.wait()


def kernel(predictions, gt_boxes, gt_labels):
    pT = predictions[0].T
    gtbT = gt_boxes.T
    glab = gt_labels.reshape(32, 1)
    const = jnp.asarray(_CONST_T)

    predsT, loss = pl.pallas_call(
        _body,
        out_shape=(
            jax.ShapeDtypeStruct((_NOUT, _NV), jnp.float32),
            jax.ShapeDtypeStruct((1, 1), jnp.float32),
        ),
        in_specs=[
            pl.BlockSpec(memory_space=pl.ANY),
            pl.BlockSpec((4, 32), lambda: (0, 0)),
            pl.BlockSpec((32, 1), lambda: (0, 0)),
            pl.BlockSpec((9, _NV), lambda: (0, 0)),
        ],
        out_specs=(
            pl.BlockSpec(memory_space=pl.ANY),
            pl.BlockSpec((1, 1), lambda: (0, 0)),
        ),
        scratch_shapes=[
            pltpu.VMEM((_NOUT, 10647), jnp.float32),
            pltpu.VMEM((_NOUT, _NV), jnp.float32),
            pltpu.SemaphoreType.DMA,
            pltpu.SemaphoreType.DMA,
        ],
    )(pT, gtbT, glab, const)
    return predsT.T, loss.reshape(())
